# baseline (speedup 1.0000x reference)
import jax
import jax.numpy as jnp
from jax.experimental import pallas as pl
from jax.experimental.pallas import tpu as pltpu

_BM1 = 200
_BM2 = 1000


def _pass1_kernel(x_ref, a_ref, w1_ref, w2_ref, abf_ref, hw2_ref, xw1_s):
    p = pl.program_id(0)
    i = pl.program_id(1)
    bm = x_ref.shape[0]

    @pl.when(p == 0)
    def _phase0():
        xw1_s[pl.ds(i * bm, bm), :] = jax.lax.dot_general(
            x_ref[...].astype(jnp.bfloat16),
            w1_ref[...],
            (((1,), (1,)), ((), ())),
            preferred_element_type=jnp.float32,
        ).astype(jnp.bfloat16)

    @pl.when(p == 1)
    def _phase1():
        a = a_ref[...].astype(jnp.bfloat16)
        abf_ref[...] = a
        acc = jnp.dot(a, xw1_s[...], preferred_element_type=jnp.float32)
        acc = jnp.maximum(acc, 0.0).astype(jnp.bfloat16)
        hw2_ref[...] = jax.lax.dot_general(
            acc,
            w2_ref[...],
            (((1,), (1,)), ((), ())),
            preferred_element_type=jnp.float32,
        ).astype(jnp.bfloat16)


def _pass2_kernel(a_ref, h_ref, o_ref):
    acc = jnp.dot(a_ref[...], h_ref[...], preferred_element_type=jnp.float32)
    m = jnp.max(acc, axis=1, keepdims=True)
    lse = jnp.log(jnp.sum(jnp.exp(acc - m), axis=1, keepdims=True))
    o_ref[...] = acc - m - lse


def kernel(x, adjacency, W1, W2):
    n, d_in = x.shape
    d_h = W1.shape[0]
    d_out = W2.shape[0]
    bm1 = min(_BM1, n)
    bm2 = min(_BM2, n)
    num_1 = n // bm1
    num_2 = n // bm2

    w1_b = W1.astype(jnp.bfloat16)
    w2_b = W2.astype(jnp.bfloat16)

    def x_map(p, i):
        return (jnp.where(p == 0, i, num_1 - 1), 0)

    def a_map(p, i):
        return (jnp.where(p == 0, 0, i), 0)

    def o1_map(p, i):
        return (jnp.where(p == 0, 0, i), 0)

    a_bf, hw2 = pl.pallas_call(
        _pass1_kernel,
        grid=(2, num_1),
        in_specs=[
            pl.BlockSpec((bm1, d_in), x_map),
            pl.BlockSpec((bm1, n), a_map),
            pl.BlockSpec((d_h, d_in), lambda p, i: (0, 0)),
            pl.BlockSpec((d_out, d_h), lambda p, i: (0, 0)),
        ],
        out_specs=[
            pl.BlockSpec((bm1, n), o1_map),
            pl.BlockSpec((bm1, d_out), o1_map),
        ],
        out_shape=[
            jax.ShapeDtypeStruct((n, n), jnp.bfloat16),
            jax.ShapeDtypeStruct((n, d_out), jnp.bfloat16),
        ],
        scratch_shapes=[
            pltpu.VMEM((n, d_h), jnp.bfloat16),
        ],
        compiler_params=pltpu.CompilerParams(
            dimension_semantics=("arbitrary", "arbitrary"),
        ),
    )(x, adjacency, w1_b, w2_b)

    out = pl.pallas_call(
        _pass2_kernel,
        grid=(num_2,),
        in_specs=[
            pl.BlockSpec((bm2, n), lambda i: (i, 0)),
            pl.BlockSpec((n, d_out), lambda i: (0, 0)),
        ],
        out_specs=pl.BlockSpec((bm2, d_out), lambda i: (i, 0)),
        out_shape=jax.ShapeDtypeStruct((n, d_out), jnp.float32),
        compiler_params=pltpu.CompilerParams(
            dimension_semantics=("arbitrary",),
        ),
    )(a_bf, hw2)

    return out
